# baseline (speedup 1.0000x reference)
import jax
import jax.numpy as jnp
from jax import lax
from jax.experimental import pallas as pl
from jax.experimental.pallas import tpu as pltpu
from jax.experimental.pallas import tpu_sc as plsc

N = 10000
E = 320000
DIN = 128
DHID = 256
DOUT = 64

NC = 2
NS = 16
NW = NC * NS
NP = 10016
RPT = NP // NS
DEGW = 8

B = 128
NB = 80
EPAD = NW * NB * B


def _seg_sum_kernel(D, with_deg):
    assert NB % 2 == 0 and RPT > B
    mesh = plsc.VectorSubcoreMesh(core_axis_name="c", subcore_axis_name="s")
    out_type = [jax.ShapeDtypeStruct((NC, NP, D), jnp.float32)]
    scratch = [
        pltpu.VMEM_SHARED((NP, D), jnp.float32),
        pltpu.VMEM((B,), jnp.int32),
        pltpu.VMEM((B,), jnp.int32),
        pltpu.VMEM((B,), jnp.int32),
        pltpu.VMEM((B,), jnp.int32),
        pltpu.VMEM((B, D), jnp.float32),
        pltpu.VMEM((B, D), jnp.float32),
        pltpu.SemaphoreType.DMA,
        pltpu.SemaphoreType.DMA,
        pltpu.SemaphoreType.DMA,
    ]
    if with_deg:
        out_type.append(jax.ShapeDtypeStruct((NC, NP, DEGW), jnp.float32))
        scratch += [
            pltpu.VMEM_SHARED((NP, DEGW), jnp.float32),
            pltpu.VMEM((B, DEGW), jnp.float32),
            pltpu.VMEM((B, DEGW), jnp.float32),
        ]
    nfull, rem = divmod(RPT, B)

    def body(*refs):
        if with_deg:
            (table_hbm, src_hbm, dst_hbm, zer_hbm, one_hbm, z8_hbm,
             agg_out, deg_out,
             agg_sh, idx_s0, idx_d0, idx_s1, idx_d1, rows0, rows1,
             sem_g, sem_s, sem_i, deg_sh, one, b8) = refs
        else:
            (table_hbm, src_hbm, dst_hbm, zer_hbm,
             agg_out,
             agg_sh, idx_s0, idx_d0, idx_s1, idx_d1, rows0, rows1,
             sem_g, sem_s, sem_i) = refs
        c = lax.axis_index("c")
        s = lax.axis_index("s")
        wid = c * NS + s
        r0 = s * RPT
        idx_s = (idx_s0, idx_s1)
        idx_d = (idx_d0, idx_d1)
        rows = (rows0, rows1)

        pltpu.sync_copy(zer_hbm, rows0)
        if with_deg:
            pltpu.sync_copy(one_hbm, one)
            pltpu.sync_copy(z8_hbm, b8)

        for k in range(nfull):
            pltpu.sync_copy(rows0, agg_sh.at[pl.ds(r0 + k * B, B)])
            if with_deg:
                pltpu.sync_copy(b8, deg_sh.at[pl.ds(r0 + k * B, B)])
        if rem:
            r = r0 + nfull * B
            pltpu.sync_copy(rows0.at[pl.ds(0, rem)], agg_sh.at[pl.ds(r, rem)])
            if with_deg:
                pltpu.sync_copy(b8.at[pl.ds(0, rem)], deg_sh.at[pl.ds(r, rem)])
        plsc.subcore_barrier()

        def i_issue(j, p):
            jc = jnp.minimum(j, NB - 1)
            pltpu.async_copy(src_hbm.at[wid, jc], idx_s[p], sem_i)
            pltpu.async_copy(dst_hbm.at[wid, jc], idx_d[p], sem_i)

        def i_wait(p):
            pltpu.make_async_copy(src_hbm.at[wid, 0], idx_s[p], sem_i).wait()
            pltpu.make_async_copy(dst_hbm.at[wid, 0], idx_d[p], sem_i).wait()

        def g_issue(p):
            pltpu.async_copy(table_hbm.at[idx_s[p]], rows[p], sem_g)

        def g_wait(p):
            pltpu.make_async_copy(table_hbm.at[idx_s[p]], rows[p],
                                  sem_g).wait()

        def s_do(p):
            if with_deg:
                d2 = pltpu.async_copy(one, deg_sh.at[idx_d[p]], sem_s,
                                      add=True)
            pltpu.sync_copy(rows[p], agg_sh.at[idx_d[p]], add=True)
            if with_deg:
                d2.wait()

        pltpu.sync_copy(src_hbm.at[wid, 0], idx_s0)
        pltpu.sync_copy(dst_hbm.at[wid, 0], idx_d0)
        g_issue(0)
        i_issue(1, 1)

        def pair(jj, carry):
            j0 = jj * 2
            g_wait(0)
            i_wait(1)
            g_issue(1)
            s_do(0)
            i_issue(j0 + 2, 0)
            g_wait(1)
            i_wait(0)
            g_issue(0)
            s_do(1)
            i_issue(j0 + 3, 1)
            return carry

        lax.fori_loop(0, NB // 2, pair, 0)
        g_wait(0)
        i_wait(1)
        plsc.subcore_barrier()

        for k in range(nfull):
            r = r0 + k * B
            pltpu.sync_copy(agg_sh.at[pl.ds(r, B)], rows0)
            pltpu.sync_copy(rows0, agg_out.at[c, pl.ds(r, B)])
            if with_deg:
                pltpu.sync_copy(deg_sh.at[pl.ds(r, B)], b8)
                pltpu.sync_copy(b8, deg_out.at[c, pl.ds(r, B)])
        if rem:
            r = r0 + nfull * B
            pltpu.sync_copy(agg_sh.at[pl.ds(r, rem)], rows0.at[pl.ds(0, rem)])
            pltpu.sync_copy(rows0.at[pl.ds(0, rem)],
                            agg_out.at[c, pl.ds(r, rem)])
            if with_deg:
                pltpu.sync_copy(deg_sh.at[pl.ds(r, rem)], b8.at[pl.ds(0, rem)])
                pltpu.sync_copy(b8.at[pl.ds(0, rem)],
                                deg_out.at[c, pl.ds(r, rem)])

    return pl.kernel(body, out_type=tuple(out_type), mesh=mesh,
                     scratch_types=tuple(scratch),
                     compiler_params=pltpu.CompilerParams(
                         use_tc_tiling_on_sc=False))


_seg128 = _seg_sum_kernel(DIN, with_deg=True)
_seg64 = _seg_sum_kernel(DOUT, with_deg=False)


def _tc1_body(x_ref, agg_ref, deg_ref, w1s_ref, w1n_ref, b1_ref,
              w2s_ref, w2n_ref, b2_ref, z_ref, s2_ref):
    x = x_ref[...]
    agg = agg_ref[0] + agg_ref[1]
    dg = deg_ref[0, :, 0:1] + deg_ref[1, :, 0:1]
    inv = 1.0 / jnp.maximum(dg, 1.0)
    hn = agg * inv
    h = jnp.dot(x, w1s_ref[...], preferred_element_type=jnp.float32)
    h = h + jnp.dot(hn, w1n_ref[...], preferred_element_type=jnp.float32)
    h = h + b1_ref[...]
    z_ref[...] = jnp.dot(h, w2n_ref[...], preferred_element_type=jnp.float32)
    s2_ref[...] = (jnp.dot(h, w2s_ref[...], preferred_element_type=jnp.float32)
                   + b2_ref[...])


def _tc2_body(s2_ref, agg2_ref, deg_ref, o_ref):
    dg = deg_ref[0, :, 0:1] + deg_ref[1, :, 0:1]
    inv = 1.0 / jnp.maximum(dg, 1.0)
    o_ref[...] = s2_ref[...] + (agg2_ref[0] + agg2_ref[1]) * inv


_R = 1000


def _tc1(x, agg1, deg, w1sT, w1nT, b1r, w2sT, w2nT, b2r):
    grid = (N // _R,)
    return pl.pallas_call(
        _tc1_body,
        grid=grid,
        in_specs=[
            pl.BlockSpec((_R, DIN), lambda i: (i, 0)),
            pl.BlockSpec((NC, _R, DIN), lambda i: (0, i, 0)),
            pl.BlockSpec((NC, _R, DEGW), lambda i: (0, i, 0)),
            pl.BlockSpec((DIN, DHID), lambda i: (0, 0)),
            pl.BlockSpec((DIN, DHID), lambda i: (0, 0)),
            pl.BlockSpec((1, DHID), lambda i: (0, 0)),
            pl.BlockSpec((DHID, DOUT), lambda i: (0, 0)),
            pl.BlockSpec((DHID, DOUT), lambda i: (0, 0)),
            pl.BlockSpec((1, DOUT), lambda i: (0, 0)),
        ],
        out_specs=[
            pl.BlockSpec((_R, DOUT), lambda i: (i, 0)),
            pl.BlockSpec((_R, DOUT), lambda i: (i, 0)),
        ],
        out_shape=[
            jax.ShapeDtypeStruct((N, DOUT), jnp.float32),
            jax.ShapeDtypeStruct((N, DOUT), jnp.float32),
        ],
    )(x, agg1, deg, w1sT, w1nT, b1r, w2sT, w2nT, b2r)


def _tc2(s2, agg2, deg):
    grid = (N // _R,)
    return pl.pallas_call(
        _tc2_body,
        grid=grid,
        in_specs=[
            pl.BlockSpec((_R, DOUT), lambda i: (i, 0)),
            pl.BlockSpec((NC, _R, DOUT), lambda i: (0, i, 0)),
            pl.BlockSpec((NC, _R, DEGW), lambda i: (0, i, 0)),
        ],
        out_specs=pl.BlockSpec((_R, DOUT), lambda i: (i, 0)),
        out_shape=jax.ShapeDtypeStruct((N, DOUT), jnp.float32),
    )(s2, agg2, deg)


def kernel(features, edge_index, W1_self, W1_neigh, b1, W2_self, W2_neigh, b2):
    src = edge_index[0].astype(jnp.int32)
    dst = edge_index[1].astype(jnp.int32)
    pad = EPAD - E
    s3 = jnp.concatenate([src, jnp.zeros((pad,), jnp.int32)]).reshape(
        NW, NB, B)
    d3 = jnp.concatenate([dst, jnp.full((pad,), N, jnp.int32)]).reshape(
        NW, NB, B)

    zer1 = jnp.zeros((B, DIN), jnp.float32)
    zer2 = jnp.zeros((B, DOUT), jnp.float32)
    one8 = jnp.ones((B, DEGW), jnp.float32)
    zer8 = jnp.zeros((B, DEGW), jnp.float32)

    aggp, degp = _seg128(features, s3, d3, zer1, one8, zer8)
    agg1 = aggp[:, :N]
    deg = degp[:, :N]

    z, s2 = _tc1(features, agg1, deg, W1_self.T, W1_neigh.T, b1[None],
                 W2_self.T, W2_neigh.T, b2[None])

    (agg2p,) = _seg64(z, s3, d3, zer2)
    out = _tc2(s2, agg2p[:, :N], deg)
    return out


# speedup vs baseline: 1.2562x; 1.2562x over previous
import jax
import jax.numpy as jnp
from jax import lax
from jax.experimental import pallas as pl
from jax.experimental.pallas import tpu as pltpu
from jax.experimental.pallas import tpu_sc as plsc

N = 10000
E = 320000
DIN = 128
DHID = 256
DOUT = 64

NC = 2
NS = 16
NW = NC * NS
NP = 10016
RPT = NP // NS
DEGW = 8

B1, NB1 = 64, 158
B2, NB2 = 128, 80
EPAD1 = NW * NB1 * B1
EPAD2 = NW * NB2 * B2


def _seg_sum_kernel(D, B, NB, with_deg):
    assert NB % 2 == 0 and RPT > B
    mesh = plsc.VectorSubcoreMesh(core_axis_name="c", subcore_axis_name="s")
    out_type = [jax.ShapeDtypeStruct((NC, NP, D), jnp.float32)]
    scratch = [
        pltpu.VMEM_SHARED((NP, D), jnp.float32),
        pltpu.VMEM((NB, B), jnp.int32),
        pltpu.VMEM((NB, B), jnp.int32),
        pltpu.VMEM((B, D), jnp.float32),
        pltpu.VMEM((B, D), jnp.float32),
        pltpu.SemaphoreType.DMA,
        pltpu.SemaphoreType.DMA,
    ]
    if with_deg:
        out_type.append(jax.ShapeDtypeStruct((NC, NP, DEGW), jnp.float32))
        scratch += [
            pltpu.VMEM_SHARED((NP, DEGW), jnp.float32),
            pltpu.VMEM((B, DEGW), jnp.float32),
            pltpu.VMEM((B, DEGW), jnp.float32),
        ]
    nfull, rem = divmod(RPT, B)

    def body(*refs):
        if with_deg:
            (table_hbm, src_hbm, dst_hbm, zer_hbm, one_hbm, z8_hbm,
             agg_out, deg_out,
             agg_sh, idx_s_all, idx_d_all, rows0, rows1, sem_g, sem_s,
             deg_sh, one, b8) = refs
        else:
            (table_hbm, src_hbm, dst_hbm, zer_hbm,
             agg_out,
             agg_sh, idx_s_all, idx_d_all, rows0, rows1, sem_g, sem_s) = refs
        c = lax.axis_index("c")
        s = lax.axis_index("s")
        wid = c * NS + s
        r0 = s * RPT

        pltpu.sync_copy(src_hbm.at[wid], idx_s_all)
        pltpu.sync_copy(dst_hbm.at[wid], idx_d_all)
        pltpu.sync_copy(zer_hbm, rows0)
        if with_deg:
            pltpu.sync_copy(one_hbm, one)
            pltpu.sync_copy(z8_hbm, b8)

        for k in range(nfull):
            pltpu.sync_copy(rows0, agg_sh.at[pl.ds(r0 + k * B, B)])
            if with_deg:
                pltpu.sync_copy(b8, deg_sh.at[pl.ds(r0 + k * B, B)])
        if rem:
            r = r0 + nfull * B
            pltpu.sync_copy(rows0.at[pl.ds(0, rem)], agg_sh.at[pl.ds(r, rem)])
            if with_deg:
                pltpu.sync_copy(b8.at[pl.ds(0, rem)], deg_sh.at[pl.ds(r, rem)])
        plsc.subcore_barrier()

        def g_issue(j, buf):
            pltpu.async_copy(table_hbm.at[idx_s_all.at[j]], buf, sem_g)

        def g_wait(j, buf):
            pltpu.make_async_copy(table_hbm.at[idx_s_all.at[j]], buf,
                                  sem_g).wait()

        def s_do(j, buf):
            if with_deg:
                d2 = pltpu.async_copy(one, deg_sh.at[idx_d_all.at[j]], sem_s,
                                      add=True)
            pltpu.sync_copy(buf, agg_sh.at[idx_d_all.at[j]], add=True)
            if with_deg:
                d2.wait()

        g_issue(0, rows0)

        def pair(jj, carry):
            j0 = jj * 2
            g_wait(j0, rows0)
            g_issue(j0 + 1, rows1)
            s_do(j0, rows0)
            g_wait(j0 + 1, rows1)
            g_issue(j0 + 2, rows0)
            s_do(j0 + 1, rows1)
            return carry

        lax.fori_loop(0, NB // 2 - 1, pair, 0)
        j0 = NB - 2
        g_wait(j0, rows0)
        g_issue(j0 + 1, rows1)
        s_do(j0, rows0)
        g_wait(j0 + 1, rows1)
        s_do(j0 + 1, rows1)
        plsc.subcore_barrier()

        pltpu.sync_copy(agg_sh.at[pl.ds(r0, RPT)], agg_out.at[c, pl.ds(r0, RPT)])
        if with_deg:
            pltpu.sync_copy(deg_sh.at[pl.ds(r0, RPT)],
                            deg_out.at[c, pl.ds(r0, RPT)])

    return pl.kernel(body, out_type=tuple(out_type), mesh=mesh,
                     scratch_types=tuple(scratch),
                     compiler_params=pltpu.CompilerParams(
                         use_tc_tiling_on_sc=False))


_seg128 = _seg_sum_kernel(DIN, B1, NB1, with_deg=True)
_seg64 = _seg_sum_kernel(DOUT, B2, NB2, with_deg=False)


def _tc1_body(x_ref, agg_ref, deg_ref, w1s_ref, w1n_ref, b1_ref,
              w2s_ref, w2n_ref, b2_ref, z_ref, s2_ref):
    x = x_ref[...]
    agg = agg_ref[0] + agg_ref[1]
    dg = deg_ref[0, :, 0:1] + deg_ref[1, :, 0:1]
    inv = 1.0 / jnp.maximum(dg, 1.0)
    hn = agg * inv
    h = jnp.dot(x, w1s_ref[...], preferred_element_type=jnp.float32)
    h = h + jnp.dot(hn, w1n_ref[...], preferred_element_type=jnp.float32)
    h = h + b1_ref[...]
    z_ref[...] = jnp.dot(h, w2n_ref[...], preferred_element_type=jnp.float32)
    s2_ref[...] = (jnp.dot(h, w2s_ref[...], preferred_element_type=jnp.float32)
                   + b2_ref[...])


def _tc2_body(s2_ref, agg2_ref, deg_ref, o_ref):
    dg = deg_ref[0, :, 0:1] + deg_ref[1, :, 0:1]
    inv = 1.0 / jnp.maximum(dg, 1.0)
    o_ref[...] = s2_ref[...] + (agg2_ref[0] + agg2_ref[1]) * inv


_R = 1000


def _tc1(x, agg1, deg, w1sT, w1nT, b1r, w2sT, w2nT, b2r):
    grid = (N // _R,)
    return pl.pallas_call(
        _tc1_body,
        grid=grid,
        in_specs=[
            pl.BlockSpec((_R, DIN), lambda i: (i, 0)),
            pl.BlockSpec((NC, _R, DIN), lambda i: (0, i, 0)),
            pl.BlockSpec((NC, _R, DEGW), lambda i: (0, i, 0)),
            pl.BlockSpec((DIN, DHID), lambda i: (0, 0)),
            pl.BlockSpec((DIN, DHID), lambda i: (0, 0)),
            pl.BlockSpec((1, DHID), lambda i: (0, 0)),
            pl.BlockSpec((DHID, DOUT), lambda i: (0, 0)),
            pl.BlockSpec((DHID, DOUT), lambda i: (0, 0)),
            pl.BlockSpec((1, DOUT), lambda i: (0, 0)),
        ],
        out_specs=[
            pl.BlockSpec((_R, DOUT), lambda i: (i, 0)),
            pl.BlockSpec((_R, DOUT), lambda i: (i, 0)),
        ],
        out_shape=[
            jax.ShapeDtypeStruct((N, DOUT), jnp.float32),
            jax.ShapeDtypeStruct((N, DOUT), jnp.float32),
        ],
    )(x, agg1, deg, w1sT, w1nT, b1r, w2sT, w2nT, b2r)


def _tc2(s2, agg2, deg):
    grid = (N // _R,)
    return pl.pallas_call(
        _tc2_body,
        grid=grid,
        in_specs=[
            pl.BlockSpec((_R, DOUT), lambda i: (i, 0)),
            pl.BlockSpec((NC, _R, DOUT), lambda i: (0, i, 0)),
            pl.BlockSpec((NC, _R, DEGW), lambda i: (0, i, 0)),
        ],
        out_specs=pl.BlockSpec((_R, DOUT), lambda i: (i, 0)),
        out_shape=jax.ShapeDtypeStruct((N, DOUT), jnp.float32),
    )(s2, agg2, deg)


def _pad_edges(src, dst, epad, nb, b):
    pad = epad - E
    src_p = jnp.concatenate([src, jnp.zeros((pad,), jnp.int32)])
    dst_p = jnp.concatenate([dst, jnp.full((pad,), N, jnp.int32)])
    return src_p.reshape(NW, nb, b), dst_p.reshape(NW, nb, b)


def kernel(features, edge_index, W1_self, W1_neigh, b1, W2_self, W2_neigh, b2):
    src = edge_index[0].astype(jnp.int32)
    dst = edge_index[1].astype(jnp.int32)
    s1, d1 = _pad_edges(src, dst, EPAD1, NB1, B1)
    s2e, d2e = _pad_edges(src, dst, EPAD2, NB2, B2)

    zer1 = jnp.zeros((B1, DIN), jnp.float32)
    zer2 = jnp.zeros((B2, DOUT), jnp.float32)
    one8 = jnp.ones((B1, DEGW), jnp.float32)
    zer8 = jnp.zeros((B1, DEGW), jnp.float32)

    aggp, degp = _seg128(features, s1, d1, zer1, one8, zer8)
    agg1 = aggp[:, :N]
    deg = degp[:, :N]

    z, s2 = _tc1(features, agg1, deg, W1_self.T, W1_neigh.T, b1[None],
                 W2_self.T, W2_neigh.T, b2[None])

    (agg2p,) = _seg64(z, s2e, d2e, zer2)
    out = _tc2(s2, agg2p[:, :N], deg)
    return out


# speedup vs baseline: 1.3065x; 1.0400x over previous
import jax
import jax.numpy as jnp
from jax import lax
from jax.experimental import pallas as pl
from jax.experimental.pallas import tpu as pltpu
from jax.experimental.pallas import tpu_sc as plsc

N = 10000
E = 320000
DIN = 128
DHID = 256
DOUT = 64

NC = 2
NS = 16
NW = NC * NS
NP = 10016
RPT = NP // NS
DEGW = 8

B1, NB1 = 64, 158
B2, NB2 = 128, 80
EPAD1 = NW * NB1 * B1
EPAD2 = NW * NB2 * B2


def _seg_sum_kernel(D, B, NB, with_deg):
    assert NB % 2 == 0 and RPT > B
    mesh = plsc.VectorSubcoreMesh(core_axis_name="c", subcore_axis_name="s")
    out_type = [jax.ShapeDtypeStruct((NC, NP, D), jnp.float32)]
    scratch = [
        pltpu.VMEM_SHARED((NP, D), jnp.float32),
        pltpu.VMEM((NB, B), jnp.int32),
        pltpu.VMEM((NB, B), jnp.int32),
        pltpu.VMEM((B, D), jnp.float32),
        pltpu.VMEM((B, D), jnp.float32),
        pltpu.SemaphoreType.DMA,
        pltpu.SemaphoreType.DMA,
    ]
    if with_deg:
        out_type.append(jax.ShapeDtypeStruct((NC, NP, DEGW), jnp.float32))
        scratch += [
            pltpu.VMEM_SHARED((NP, DEGW), jnp.float32),
            pltpu.VMEM((B, DEGW), jnp.float32),
            pltpu.VMEM((B, DEGW), jnp.float32),
        ]
    nfull, rem = divmod(RPT, B)

    def body(*refs):
        if with_deg:
            (table_hbm, src_hbm, dst_hbm, zer_hbm, one_hbm, z8_hbm,
             agg_out, deg_out,
             agg_sh, idx_s_all, idx_d_all, rows0, rows1, sem_g, sem_s,
             deg_sh, one, b8) = refs
        else:
            (table_hbm, src_hbm, dst_hbm, zer_hbm,
             agg_out,
             agg_sh, idx_s_all, idx_d_all, rows0, rows1, sem_g, sem_s) = refs
        c = lax.axis_index("c")
        s = lax.axis_index("s")
        wid = c * NS + s
        r0 = s * RPT

        pltpu.sync_copy(src_hbm.at[wid], idx_s_all)
        pltpu.sync_copy(dst_hbm.at[wid], idx_d_all)
        pltpu.sync_copy(zer_hbm, rows0)
        if with_deg:
            pltpu.sync_copy(one_hbm, one)
            pltpu.sync_copy(z8_hbm, b8)

        for k in range(nfull):
            pltpu.sync_copy(rows0, agg_sh.at[pl.ds(r0 + k * B, B)])
            if with_deg:
                pltpu.sync_copy(b8, deg_sh.at[pl.ds(r0 + k * B, B)])
        if rem:
            r = r0 + nfull * B
            pltpu.sync_copy(rows0.at[pl.ds(0, rem)], agg_sh.at[pl.ds(r, rem)])
            if with_deg:
                pltpu.sync_copy(b8.at[pl.ds(0, rem)], deg_sh.at[pl.ds(r, rem)])
        plsc.subcore_barrier()

        def g_issue(j, buf):
            pltpu.async_copy(table_hbm.at[idx_s_all.at[j]], buf, sem_g)

        def g_wait(j, buf):
            pltpu.make_async_copy(table_hbm.at[idx_s_all.at[j]], buf,
                                  sem_g).wait()

        def s_do(j, buf):
            if with_deg:
                d2 = pltpu.async_copy(one, deg_sh.at[idx_d_all.at[j]], sem_s,
                                      add=True)
            pltpu.sync_copy(buf, agg_sh.at[idx_d_all.at[j]], add=True)
            if with_deg:
                d2.wait()

        g_issue(0, rows0)

        def pair(jj, carry):
            j0 = jj * 2
            g_wait(j0, rows0)
            g_issue(j0 + 1, rows1)
            s_do(j0, rows0)
            g_wait(j0 + 1, rows1)
            g_issue(j0 + 2, rows0)
            s_do(j0 + 1, rows1)
            return carry

        lax.fori_loop(0, NB // 2 - 1, pair, 0)
        j0 = NB - 2
        g_wait(j0, rows0)
        g_issue(j0 + 1, rows1)
        s_do(j0, rows0)
        g_wait(j0 + 1, rows1)
        s_do(j0 + 1, rows1)
        plsc.subcore_barrier()

        pltpu.sync_copy(agg_sh.at[pl.ds(r0, RPT)], agg_out.at[c, pl.ds(r0, RPT)])
        if with_deg:
            pltpu.sync_copy(deg_sh.at[pl.ds(r0, RPT)],
                            deg_out.at[c, pl.ds(r0, RPT)])

    return pl.kernel(body, out_type=tuple(out_type), mesh=mesh,
                     scratch_types=tuple(scratch),
                     compiler_params=pltpu.CompilerParams(
                         use_tc_tiling_on_sc=False))


_seg128 = _seg_sum_kernel(DIN, B1, NB1, with_deg=True)
_seg64 = _seg_sum_kernel(DOUT, B2, NB2, with_deg=False)


def _wprep_body(w1s_ref, w1n_ref, w2s_ref, w2n_ref, b1_ref, b2_ref,
                msu_ref, mss_ref, mc_ref, c2n_ref, c2s_ref):
    w1s, w1n = w1s_ref[...], w1n_ref[...]
    w2s, w2n = w2s_ref[...], w2n_ref[...]
    dot = lambda a, b: jnp.dot(a, b, preferred_element_type=jnp.float32)
    msu_ref[...] = dot(w1n, w2n)
    mss_ref[...] = dot(w1s, w2s)
    mc_ref[...] = dot(w1n, w2s) + dot(w1s, w2n)
    c2n_ref[...] = dot(b1_ref[...], w2n)
    c2s_ref[...] = dot(b1_ref[...], w2s) + b2_ref[...]


def _wprep(w1sT, w1nT, w2sT, w2nT, b1r, b2r):
    full = lambda s: pl.BlockSpec(s, lambda: tuple(0 for _ in s))
    m = jax.ShapeDtypeStruct((DIN, DOUT), jnp.float32)
    v = jax.ShapeDtypeStruct((1, DOUT), jnp.float32)
    return pl.pallas_call(
        _wprep_body,
        in_specs=[full((DIN, DHID)), full((DIN, DHID)),
                  full((DHID, DOUT)), full((DHID, DOUT)),
                  full((1, DHID)), full((1, DOUT))],
        out_specs=[full((DIN, DOUT))] * 3 + [full((1, DOUT))] * 2,
        out_shape=[m, m, m, v, v],
    )(w1sT, w1nT, w2sT, w2nT, b1r, b2r)


def _inv_deg(deg_ref):
    dg = deg_ref[0, :, 0:1] + deg_ref[1, :, 0:1]
    return dg, 1.0 / jnp.maximum(dg, 1.0)


def _tcmid_body(agg_ref, deg_ref, msu_ref, u_ref):
    _, inv = _inv_deg(deg_ref)
    agg = agg_ref[0] + agg_ref[1]
    u_ref[...] = jnp.dot(agg, msu_ref[...],
                         preferred_element_type=jnp.float32) * inv


def _tcpar_body(x_ref, agg_ref, deg_ref, mss_ref, mc_ref, c2n_ref, c2s_ref,
                b_ref):
    dg, inv = _inv_deg(deg_ref)
    agg = agg_ref[0] + agg_ref[1]
    b = jnp.dot(x_ref[...], mss_ref[...], preferred_element_type=jnp.float32)
    b = b + jnp.dot(agg, mc_ref[...], preferred_element_type=jnp.float32) * inv
    b_ref[...] = b + c2s_ref[...] + jnp.where(dg > 0.0, c2n_ref[...], 0.0)


def _tcfin_body(b_ref, agg2_ref, deg_ref, o_ref):
    _, inv = _inv_deg(deg_ref)
    o_ref[...] = b_ref[...] + (agg2_ref[0] + agg2_ref[1]) * inv


_RP = 2504
_R = 2000


def _tcmid(aggp, degp, msu):
    return pl.pallas_call(
        _tcmid_body,
        grid=(NP // _RP,),
        in_specs=[
            pl.BlockSpec((NC, _RP, DIN), lambda i: (0, i, 0)),
            pl.BlockSpec((NC, _RP, DEGW), lambda i: (0, i, 0)),
            pl.BlockSpec((DIN, DOUT), lambda i: (0, 0)),
        ],
        out_specs=pl.BlockSpec((_RP, DOUT), lambda i: (i, 0)),
        out_shape=jax.ShapeDtypeStruct((NP, DOUT), jnp.float32),
    )(aggp, degp, msu)


def _tcpar(x, aggp, degp, mss, mc, c2n, c2s):
    return pl.pallas_call(
        _tcpar_body,
        grid=(N // _R,),
        in_specs=[
            pl.BlockSpec((_R, DIN), lambda i: (i, 0)),
            pl.BlockSpec((NC, _R, DIN), lambda i: (0, i, 0)),
            pl.BlockSpec((NC, _R, DEGW), lambda i: (0, i, 0)),
            pl.BlockSpec((DIN, DOUT), lambda i: (0, 0)),
            pl.BlockSpec((DIN, DOUT), lambda i: (0, 0)),
            pl.BlockSpec((1, DOUT), lambda i: (0, 0)),
            pl.BlockSpec((1, DOUT), lambda i: (0, 0)),
        ],
        out_specs=pl.BlockSpec((_R, DOUT), lambda i: (i, 0)),
        out_shape=jax.ShapeDtypeStruct((N, DOUT), jnp.float32),
    )(x, aggp, degp, mss, mc, c2n, c2s)


def _tcfin(base, agg2p, degp):
    return pl.pallas_call(
        _tcfin_body,
        grid=(N // _R,),
        in_specs=[
            pl.BlockSpec((_R, DOUT), lambda i: (i, 0)),
            pl.BlockSpec((NC, _R, DOUT), lambda i: (0, i, 0)),
            pl.BlockSpec((NC, _R, DEGW), lambda i: (0, i, 0)),
        ],
        out_specs=pl.BlockSpec((_R, DOUT), lambda i: (i, 0)),
        out_shape=jax.ShapeDtypeStruct((N, DOUT), jnp.float32),
    )(base, agg2p, degp)


def _pad_edges(src, dst, epad, nb, b):
    pad = epad - E
    src_p = jnp.concatenate([src, jnp.zeros((pad,), jnp.int32)])
    dst_p = jnp.concatenate([dst, jnp.full((pad,), N, jnp.int32)])
    return src_p.reshape(NW, nb, b), dst_p.reshape(NW, nb, b)


def kernel(features, edge_index, W1_self, W1_neigh, b1, W2_self, W2_neigh, b2):
    src = edge_index[0].astype(jnp.int32)
    dst = edge_index[1].astype(jnp.int32)
    s1, d1 = _pad_edges(src, dst, EPAD1, NB1, B1)
    s2e, d2e = _pad_edges(src, dst, EPAD2, NB2, B2)

    zer1 = jnp.zeros((B1, DIN), jnp.float32)
    zer2 = jnp.zeros((B2, DOUT), jnp.float32)
    one8 = jnp.ones((B1, DEGW), jnp.float32)
    zer8 = jnp.zeros((B1, DEGW), jnp.float32)

    msu, mss, mc, c2n, c2s = _wprep(W1_self.T, W1_neigh.T, W2_self.T,
                                    W2_neigh.T, b1[None], b2[None])

    aggp, degp = _seg128(features, s1, d1, zer1, one8, zer8)
    u = _tcmid(aggp, degp, msu)
    base = _tcpar(features, aggp, degp, mss, mc, c2n, c2s)

    (agg2p,) = _seg64(u, s2e, d2e, zer2)
    return _tcfin(base, agg2p, degp)


# speedup vs baseline: 1.4516x; 1.1111x over previous
import jax
import jax.numpy as jnp
from jax import lax
from jax.experimental import pallas as pl
from jax.experimental.pallas import tpu as pltpu
from jax.experimental.pallas import tpu_sc as plsc

N = 10000
E = 320000
DIN = 128
DHID = 256
DOUT = 64

NC = 2
NS = 16
NW = NC * NS
NP = 10016
RPT = NP // NS
DEGW = 8

B1, NB1 = 64, 158
B2, NB2 = 128, 80
EPAD1 = NW * NB1 * B1
EPAD2 = NW * NB2 * B2


def _seg_sum_kernel(D, B, NB, with_deg, depth=2):
    assert NB % depth == 0 and NB // depth >= 2 and RPT > B
    mesh = plsc.VectorSubcoreMesh(core_axis_name="c", subcore_axis_name="s")
    out_type = [jax.ShapeDtypeStruct((NC, NP, D), jnp.float32)]
    scratch = [
        pltpu.VMEM_SHARED((NP, D), jnp.float32),
        pltpu.VMEM((NB, B), jnp.int32),
        pltpu.VMEM((NB, B), jnp.int32),
    ]
    scratch += [pltpu.VMEM((B, D), jnp.float32) for _ in range(depth)]
    scratch += [pltpu.SemaphoreType.DMA for _ in range(depth)]
    scratch += [pltpu.SemaphoreType.DMA]
    if with_deg:
        out_type.append(jax.ShapeDtypeStruct((NC, NP, DEGW), jnp.float32))
        scratch += [
            pltpu.VMEM_SHARED((NP, DEGW), jnp.float32),
            pltpu.VMEM((B, DEGW), jnp.float32),
            pltpu.VMEM((B, DEGW), jnp.float32),
        ]
    nfull, rem = divmod(RPT, B)

    def body(*refs):
        if with_deg:
            (table_hbm, src_hbm, dst_hbm, zer_hbm, one_hbm, z8_hbm,
             agg_out, deg_out, agg_sh, idx_s_all, idx_d_all) = refs[:11]
            bufs = refs[11:11 + depth]
            sems = refs[11 + depth:11 + 2 * depth]
            sem_s = refs[11 + 2 * depth]
            deg_sh, one, b8 = refs[12 + 2 * depth:]
        else:
            (table_hbm, src_hbm, dst_hbm, zer_hbm,
             agg_out, agg_sh, idx_s_all, idx_d_all) = refs[:8]
            bufs = refs[8:8 + depth]
            sems = refs[8 + depth:8 + 2 * depth]
            sem_s = refs[8 + 2 * depth]
        c = lax.axis_index("c")
        s = lax.axis_index("s")
        wid = c * NS + s
        r0 = s * RPT

        pltpu.sync_copy(src_hbm.at[wid], idx_s_all)
        pltpu.sync_copy(dst_hbm.at[wid], idx_d_all)
        pltpu.sync_copy(zer_hbm, bufs[0])
        if with_deg:
            pltpu.sync_copy(one_hbm, one)
            pltpu.sync_copy(z8_hbm, b8)

        for k in range(nfull):
            pltpu.sync_copy(bufs[0], agg_sh.at[pl.ds(r0 + k * B, B)])
            if with_deg:
                pltpu.sync_copy(b8, deg_sh.at[pl.ds(r0 + k * B, B)])
        if rem:
            r = r0 + nfull * B
            pltpu.sync_copy(bufs[0].at[pl.ds(0, rem)], agg_sh.at[pl.ds(r, rem)])
            if with_deg:
                pltpu.sync_copy(b8.at[pl.ds(0, rem)], deg_sh.at[pl.ds(r, rem)])
        plsc.subcore_barrier()

        def g_issue(j, t):
            pltpu.async_copy(table_hbm.at[idx_s_all.at[j]], bufs[t], sems[t])

        def g_wait(j, t):
            pltpu.make_async_copy(table_hbm.at[idx_s_all.at[j]], bufs[t],
                                  sems[t]).wait()

        def s_do(j, t):
            if with_deg:
                d2 = pltpu.async_copy(one, deg_sh.at[idx_d_all.at[j]], sem_s,
                                      add=True)
            pltpu.sync_copy(bufs[t], agg_sh.at[idx_d_all.at[j]], add=True)
            if with_deg:
                d2.wait()

        for t in range(depth):
            g_issue(t, t)

        def step(jj, carry):
            j0 = jj * depth
            for t in range(depth):
                g_wait(j0 + t, t)
                s_do(j0 + t, t)
                g_issue(j0 + t + depth, t)
            return carry

        lax.fori_loop(0, NB // depth - 1, step, 0)
        j0 = NB - depth
        for t in range(depth):
            g_wait(j0 + t, t)
            s_do(j0 + t, t)
        plsc.subcore_barrier()

        pltpu.sync_copy(agg_sh.at[pl.ds(r0, RPT)], agg_out.at[c, pl.ds(r0, RPT)])
        if with_deg:
            pltpu.sync_copy(deg_sh.at[pl.ds(r0, RPT)],
                            deg_out.at[c, pl.ds(r0, RPT)])

    return pl.kernel(body, out_type=tuple(out_type), mesh=mesh,
                     scratch_types=tuple(scratch),
                     compiler_params=pltpu.CompilerParams(
                         use_tc_tiling_on_sc=False))


_seg128 = _seg_sum_kernel(DIN, B1, NB1, with_deg=True, depth=2)
_seg64 = _seg_sum_kernel(DOUT, B2, NB2, with_deg=False, depth=4)


def _wprep_body(w1s_ref, w1n_ref, w2s_ref, w2n_ref, b1_ref, b2_ref,
                msu_ref, mss_ref, mc_ref, c2n_ref, c2s_ref):
    w1s, w1n = w1s_ref[...], w1n_ref[...]
    w2s, w2n = w2s_ref[...], w2n_ref[...]
    dot = lambda a, b: jnp.dot(a, b, preferred_element_type=jnp.float32)
    msu_ref[...] = dot(w1n, w2n)
    mss_ref[...] = dot(w1s, w2s)
    mc_ref[...] = dot(w1n, w2s) + dot(w1s, w2n)
    c2n_ref[...] = dot(b1_ref[...], w2n)
    c2s_ref[...] = dot(b1_ref[...], w2s) + b2_ref[...]


def _wprep(w1sT, w1nT, w2sT, w2nT, b1r, b2r):
    full = lambda s: pl.BlockSpec(s, lambda: tuple(0 for _ in s))
    m = jax.ShapeDtypeStruct((DIN, DOUT), jnp.float32)
    v = jax.ShapeDtypeStruct((1, DOUT), jnp.float32)
    return pl.pallas_call(
        _wprep_body,
        in_specs=[full((DIN, DHID)), full((DIN, DHID)),
                  full((DHID, DOUT)), full((DHID, DOUT)),
                  full((1, DHID)), full((1, DOUT))],
        out_specs=[full((DIN, DOUT))] * 3 + [full((1, DOUT))] * 2,
        out_shape=[m, m, m, v, v],
    )(w1sT, w1nT, w2sT, w2nT, b1r, b2r)


def _inv_deg(deg_ref):
    dg = deg_ref[0, :, 0:1] + deg_ref[1, :, 0:1]
    return dg, 1.0 / jnp.maximum(dg, 1.0)


def _tcmid_body(agg_ref, deg_ref, msu_ref, u_ref):
    _, inv = _inv_deg(deg_ref)
    agg = agg_ref[0] + agg_ref[1]
    u_ref[...] = jnp.dot(agg, msu_ref[...],
                         preferred_element_type=jnp.float32) * inv


def _tcpar_body(x_ref, agg_ref, deg_ref, mss_ref, mc_ref, c2n_ref, c2s_ref,
                b_ref):
    dg, inv = _inv_deg(deg_ref)
    agg = agg_ref[0] + agg_ref[1]
    b = jnp.dot(x_ref[...], mss_ref[...], preferred_element_type=jnp.float32)
    b = b + jnp.dot(agg, mc_ref[...], preferred_element_type=jnp.float32) * inv
    b_ref[...] = b + c2s_ref[...] + jnp.where(dg > 0.0, c2n_ref[...], 0.0)


def _tcfin_body(b_ref, agg2_ref, deg_ref, o_ref):
    _, inv = _inv_deg(deg_ref)
    o_ref[...] = b_ref[...] + (agg2_ref[0] + agg2_ref[1]) * inv


_RP = 2504
_R = 2000


def _tcmid(aggp, degp, msu):
    return pl.pallas_call(
        _tcmid_body,
        grid=(NP // _RP,),
        in_specs=[
            pl.BlockSpec((NC, _RP, DIN), lambda i: (0, i, 0)),
            pl.BlockSpec((NC, _RP, DEGW), lambda i: (0, i, 0)),
            pl.BlockSpec((DIN, DOUT), lambda i: (0, 0)),
        ],
        out_specs=pl.BlockSpec((_RP, DOUT), lambda i: (i, 0)),
        out_shape=jax.ShapeDtypeStruct((NP, DOUT), jnp.float32),
    )(aggp, degp, msu)


def _tcpar(x, aggp, degp, mss, mc, c2n, c2s):
    return pl.pallas_call(
        _tcpar_body,
        grid=(N // _R,),
        in_specs=[
            pl.BlockSpec((_R, DIN), lambda i: (i, 0)),
            pl.BlockSpec((NC, _R, DIN), lambda i: (0, i, 0)),
            pl.BlockSpec((NC, _R, DEGW), lambda i: (0, i, 0)),
            pl.BlockSpec((DIN, DOUT), lambda i: (0, 0)),
            pl.BlockSpec((DIN, DOUT), lambda i: (0, 0)),
            pl.BlockSpec((1, DOUT), lambda i: (0, 0)),
            pl.BlockSpec((1, DOUT), lambda i: (0, 0)),
        ],
        out_specs=pl.BlockSpec((_R, DOUT), lambda i: (i, 0)),
        out_shape=jax.ShapeDtypeStruct((N, DOUT), jnp.float32),
    )(x, aggp, degp, mss, mc, c2n, c2s)


def _tcfin(base, agg2p, degp):
    return pl.pallas_call(
        _tcfin_body,
        grid=(N // _R,),
        in_specs=[
            pl.BlockSpec((_R, DOUT), lambda i: (i, 0)),
            pl.BlockSpec((NC, _R, DOUT), lambda i: (0, i, 0)),
            pl.BlockSpec((NC, _R, DEGW), lambda i: (0, i, 0)),
        ],
        out_specs=pl.BlockSpec((_R, DOUT), lambda i: (i, 0)),
        out_shape=jax.ShapeDtypeStruct((N, DOUT), jnp.float32),
    )(base, agg2p, degp)


def _pad_edges(src, dst, epad, nb, b):
    pad = epad - E
    src_p = jnp.concatenate([src, jnp.zeros((pad,), jnp.int32)])
    dst_p = jnp.concatenate([dst, jnp.full((pad,), N, jnp.int32)])
    return src_p.reshape(NW, nb, b), dst_p.reshape(NW, nb, b)


def kernel(features, edge_index, W1_self, W1_neigh, b1, W2_self, W2_neigh, b2):
    src = edge_index[0].astype(jnp.int32)
    dst = edge_index[1].astype(jnp.int32)
    s1, d1 = _pad_edges(src, dst, EPAD1, NB1, B1)
    s2e, d2e = _pad_edges(src, dst, EPAD2, NB2, B2)

    zer1 = jnp.zeros((B1, DIN), jnp.float32)
    zer2 = jnp.zeros((B2, DOUT), jnp.float32)
    one8 = jnp.ones((B1, DEGW), jnp.float32)
    zer8 = jnp.zeros((B1, DEGW), jnp.float32)

    msu, mss, mc, c2n, c2s = _wprep(W1_self.T, W1_neigh.T, W2_self.T,
                                    W2_neigh.T, b1[None], b2[None])

    aggp, degp = _seg128(features, s1, d1, zer1, one8, zer8)
    u = _tcmid(aggp, degp, msu)
    base = _tcpar(features, aggp, degp, mss, mc, c2n, c2s)

    (agg2p,) = _seg64(u, s2e, d2e, zer2)
    return _tcfin(base, agg2p, degp)


# speedup vs baseline: 1.4577x; 1.0042x over previous
import jax
import jax.numpy as jnp
from jax import lax
from jax.experimental import pallas as pl
from jax.experimental.pallas import tpu as pltpu
from jax.experimental.pallas import tpu_sc as plsc

N = 10000
E = 320000
DIN = 128
DHID = 256
DOUT = 64

NC = 2
NS = 16
NW = NC * NS
NP = 10016
RPT = NP // NS
DEGW = 8

B1, NB1 = 64, 158
B2, NB2 = 128, 80
EPAD1 = NW * NB1 * B1
EPAD2 = NW * NB2 * B2


def _seg_sum_kernel(D, B, NB, with_deg, depth=2):
    assert NB % depth == 0 and NB // depth >= 2 and RPT > B
    mesh = plsc.VectorSubcoreMesh(core_axis_name="c", subcore_axis_name="s")
    out_type = [jax.ShapeDtypeStruct((NC, NP, D), jnp.float32)]
    scratch = [
        pltpu.VMEM_SHARED((NP, D), jnp.float32),
        pltpu.VMEM((NB, B), jnp.int32),
        pltpu.VMEM((NB, B), jnp.int32),
    ]
    scratch += [pltpu.VMEM((B, D), jnp.float32) for _ in range(depth)]
    scratch += [pltpu.SemaphoreType.DMA for _ in range(depth)]
    scratch += [pltpu.SemaphoreType.DMA]
    if with_deg:
        out_type.append(jax.ShapeDtypeStruct((NC, NP, DEGW), jnp.float32))
        scratch += [
            pltpu.VMEM_SHARED((NP, DEGW), jnp.float32),
            pltpu.VMEM((B, DEGW), jnp.float32),
            pltpu.VMEM((B, DEGW), jnp.float32),
        ]
    nfull, rem = divmod(RPT, B)

    def body(*refs):
        if with_deg:
            (table_hbm, src_hbm, dst_hbm, zer_hbm, one_hbm, z8_hbm,
             agg_out, deg_out, agg_sh, idx_s_all, idx_d_all) = refs[:11]
            bufs = refs[11:11 + depth]
            sems = refs[11 + depth:11 + 2 * depth]
            sem_s = refs[11 + 2 * depth]
            deg_sh, one, b8 = refs[12 + 2 * depth:]
        else:
            (table_hbm, src_hbm, dst_hbm, zer_hbm,
             agg_out, agg_sh, idx_s_all, idx_d_all) = refs[:8]
            bufs = refs[8:8 + depth]
            sems = refs[8 + depth:8 + 2 * depth]
            sem_s = refs[8 + 2 * depth]
        c = lax.axis_index("c")
        s = lax.axis_index("s")
        wid = c * NS + s
        r0 = s * RPT

        pltpu.sync_copy(src_hbm.at[wid], idx_s_all)
        pltpu.sync_copy(dst_hbm.at[wid], idx_d_all)
        pltpu.sync_copy(zer_hbm, bufs[0])
        if with_deg:
            pltpu.sync_copy(one_hbm, one)
            pltpu.sync_copy(z8_hbm, b8)

        for k in range(nfull):
            pltpu.sync_copy(bufs[0], agg_sh.at[pl.ds(r0 + k * B, B)])
            if with_deg:
                pltpu.sync_copy(b8, deg_sh.at[pl.ds(r0 + k * B, B)])
        if rem:
            r = r0 + nfull * B
            pltpu.sync_copy(bufs[0].at[pl.ds(0, rem)], agg_sh.at[pl.ds(r, rem)])
            if with_deg:
                pltpu.sync_copy(b8.at[pl.ds(0, rem)], deg_sh.at[pl.ds(r, rem)])
        plsc.subcore_barrier()

        def g_issue(j, t):
            pltpu.async_copy(table_hbm.at[idx_s_all.at[j]], bufs[t], sems[t])

        def g_wait(j, t):
            pltpu.make_async_copy(table_hbm.at[idx_s_all.at[j]], bufs[t],
                                  sems[t]).wait()

        def s_do(j, t):
            if with_deg:
                d2 = pltpu.async_copy(one, deg_sh.at[idx_d_all.at[j]], sem_s,
                                      add=True)
            pltpu.sync_copy(bufs[t], agg_sh.at[idx_d_all.at[j]], add=True)
            if with_deg:
                d2.wait()

        for t in range(depth):
            g_issue(t, t)

        def step(jj, carry):
            j0 = jj * depth
            for t in range(depth):
                g_wait(j0 + t, t)
                s_do(j0 + t, t)
                g_issue(j0 + t + depth, t)
            return carry

        lax.fori_loop(0, NB // depth - 1, step, 0)
        j0 = NB - depth
        for t in range(depth):
            g_wait(j0 + t, t)
            s_do(j0 + t, t)
        plsc.subcore_barrier()

        pltpu.sync_copy(agg_sh.at[pl.ds(r0, RPT)], agg_out.at[c, pl.ds(r0, RPT)])
        if with_deg:
            pltpu.sync_copy(deg_sh.at[pl.ds(r0, RPT)],
                            deg_out.at[c, pl.ds(r0, RPT)])

    return pl.kernel(body, out_type=tuple(out_type), mesh=mesh,
                     scratch_types=tuple(scratch),
                     compiler_params=pltpu.CompilerParams(
                         use_tc_tiling_on_sc=False))


_seg128 = _seg_sum_kernel(DIN, B1, NB1, with_deg=True, depth=2)
_seg64 = _seg_sum_kernel(DOUT, B2, NB2, with_deg=False, depth=8)


def _wprep_body(w1s_ref, w1n_ref, w2s_ref, w2n_ref, b1_ref, b2_ref,
                msu_ref, mss_ref, mc_ref, c2n_ref, c2s_ref):
    w1s, w1n = w1s_ref[...], w1n_ref[...]
    w2s, w2n = w2s_ref[...], w2n_ref[...]
    dot = lambda a, b: jnp.dot(a, b, preferred_element_type=jnp.float32)
    msu_ref[...] = dot(w1n, w2n)
    mss_ref[...] = dot(w1s, w2s)
    mc_ref[...] = dot(w1n, w2s) + dot(w1s, w2n)
    c2n_ref[...] = dot(b1_ref[...], w2n)
    c2s_ref[...] = dot(b1_ref[...], w2s) + b2_ref[...]


def _wprep(w1sT, w1nT, w2sT, w2nT, b1r, b2r):
    full = lambda s: pl.BlockSpec(s, lambda: tuple(0 for _ in s))
    m = jax.ShapeDtypeStruct((DIN, DOUT), jnp.float32)
    v = jax.ShapeDtypeStruct((1, DOUT), jnp.float32)
    return pl.pallas_call(
        _wprep_body,
        in_specs=[full((DIN, DHID)), full((DIN, DHID)),
                  full((DHID, DOUT)), full((DHID, DOUT)),
                  full((1, DHID)), full((1, DOUT))],
        out_specs=[full((DIN, DOUT))] * 3 + [full((1, DOUT))] * 2,
        out_shape=[m, m, m, v, v],
    )(w1sT, w1nT, w2sT, w2nT, b1r, b2r)


def _inv_deg(deg_ref):
    dg = deg_ref[0, :, 0:1] + deg_ref[1, :, 0:1]
    return dg, 1.0 / jnp.maximum(dg, 1.0)


def _tcmid_body(agg_ref, deg_ref, msu_ref, u_ref):
    _, inv = _inv_deg(deg_ref)
    agg = agg_ref[0] + agg_ref[1]
    u_ref[...] = jnp.dot(agg, msu_ref[...],
                         preferred_element_type=jnp.float32) * inv


def _tcpar_body(x_ref, agg_ref, deg_ref, mss_ref, mc_ref, c2n_ref, c2s_ref,
                b_ref):
    dg, inv = _inv_deg(deg_ref)
    agg = agg_ref[0] + agg_ref[1]
    b = jnp.dot(x_ref[...], mss_ref[...], preferred_element_type=jnp.float32)
    b = b + jnp.dot(agg, mc_ref[...], preferred_element_type=jnp.float32) * inv
    b_ref[...] = b + c2s_ref[...] + jnp.where(dg > 0.0, c2n_ref[...], 0.0)


def _tcfin_body(b_ref, agg2_ref, deg_ref, o_ref):
    _, inv = _inv_deg(deg_ref)
    o_ref[...] = b_ref[...] + (agg2_ref[0] + agg2_ref[1]) * inv


_RP = 2504
_R = 2000


def _tcmid(aggp, degp, msu):
    return pl.pallas_call(
        _tcmid_body,
        grid=(NP // _RP,),
        in_specs=[
            pl.BlockSpec((NC, _RP, DIN), lambda i: (0, i, 0)),
            pl.BlockSpec((NC, _RP, DEGW), lambda i: (0, i, 0)),
            pl.BlockSpec((DIN, DOUT), lambda i: (0, 0)),
        ],
        out_specs=pl.BlockSpec((_RP, DOUT), lambda i: (i, 0)),
        out_shape=jax.ShapeDtypeStruct((NP, DOUT), jnp.float32),
    )(aggp, degp, msu)


def _tcpar(x, aggp, degp, mss, mc, c2n, c2s):
    return pl.pallas_call(
        _tcpar_body,
        grid=(N // _R,),
        in_specs=[
            pl.BlockSpec((_R, DIN), lambda i: (i, 0)),
            pl.BlockSpec((NC, _R, DIN), lambda i: (0, i, 0)),
            pl.BlockSpec((NC, _R, DEGW), lambda i: (0, i, 0)),
            pl.BlockSpec((DIN, DOUT), lambda i: (0, 0)),
            pl.BlockSpec((DIN, DOUT), lambda i: (0, 0)),
            pl.BlockSpec((1, DOUT), lambda i: (0, 0)),
            pl.BlockSpec((1, DOUT), lambda i: (0, 0)),
        ],
        out_specs=pl.BlockSpec((_R, DOUT), lambda i: (i, 0)),
        out_shape=jax.ShapeDtypeStruct((N, DOUT), jnp.float32),
    )(x, aggp, degp, mss, mc, c2n, c2s)


def _tcfin(base, agg2p, degp):
    return pl.pallas_call(
        _tcfin_body,
        grid=(N // _R,),
        in_specs=[
            pl.BlockSpec((_R, DOUT), lambda i: (i, 0)),
            pl.BlockSpec((NC, _R, DOUT), lambda i: (0, i, 0)),
            pl.BlockSpec((NC, _R, DEGW), lambda i: (0, i, 0)),
        ],
        out_specs=pl.BlockSpec((_R, DOUT), lambda i: (i, 0)),
        out_shape=jax.ShapeDtypeStruct((N, DOUT), jnp.float32),
    )(base, agg2p, degp)


def _pad_edges(src, dst, epad, nb, b):
    pad = epad - E
    src_p = jnp.concatenate([src, jnp.zeros((pad,), jnp.int32)])
    dst_p = jnp.concatenate([dst, jnp.full((pad,), N, jnp.int32)])
    return src_p.reshape(NW, nb, b), dst_p.reshape(NW, nb, b)


def kernel(features, edge_index, W1_self, W1_neigh, b1, W2_self, W2_neigh, b2):
    src = edge_index[0].astype(jnp.int32)
    dst = edge_index[1].astype(jnp.int32)
    s1, d1 = _pad_edges(src, dst, EPAD1, NB1, B1)
    s2e, d2e = _pad_edges(src, dst, EPAD2, NB2, B2)

    zer1 = jnp.zeros((B1, DIN), jnp.float32)
    zer2 = jnp.zeros((B2, DOUT), jnp.float32)
    one8 = jnp.ones((B1, DEGW), jnp.float32)
    zer8 = jnp.zeros((B1, DEGW), jnp.float32)

    msu, mss, mc, c2n, c2s = _wprep(W1_self.T, W1_neigh.T, W2_self.T,
                                    W2_neigh.T, b1[None], b2[None])

    aggp, degp = _seg128(features, s1, d1, zer1, one8, zer8)
    u = _tcmid(aggp, degp, msu)
    base = _tcpar(features, aggp, degp, mss, mc, c2n, c2s)

    (agg2p,) = _seg64(u, s2e, d2e, zer2)
    return _tcfin(base, agg2p, degp)
